# baseline (speedup 1.0000x reference)
import jax
import jax.numpy as jnp
from jax import lax
from jax.experimental import pallas as pl
from jax.experimental.pallas import tpu as pltpu
from jax.experimental.pallas import tpu_sc as plsc

_LANES = 16
_NC, _NS = 2, 16
_NW = _NC * _NS
_ROWS = 64 * 128
_B = 8
_GROUPS = _ROWS // _B
_GPW = _GROUPS // _NW
_PRE = 8


def _fold_spec(oh, ow, kh, kw, sh, sw, ph, pw, dh, dw, ntap, L):
    assert sw == 1, "all three folds have unit output-width stride"
    Lh = (oh + 2 * ph - dh * (kh - 1) - 1) // sh + 1
    Lw = (ow + 2 * pw - dw * (kw - 1) - 1) // sw + 1
    assert ntap == kh * kw and L == Lh * Lw
    slab = ntap * L
    olen = oh * ow
    rows = [[] for _ in range(oh)]
    for ki in range(kh):
        for kj in range(kw):
            for lh in range(Lh):
                r = lh * sh + ki * dh - ph
                if r < 0 or r >= oh:
                    continue
                c0 = kj * dw - pw
                s = max(0, c0)
                e = min(Lw + c0, ow)
                if e <= s:
                    continue
                rows[r].append((((ki * kw + kj) * Lh + lh) * Lw - c0, s, e))
    vecs = []
    for r in range(oh):
        for k in range(0, ow, _LANES):
            contribs = []
            for src0, s, e in rows[r]:
                a = max(s - k, 0)
                b = min(e - k, _LANES)
                if b > a:
                    contribs.append((src0 + k, a, b))
            assert contribs
            vecs.append((r * ow + k, contribs))
    tail = olen - vecs[-1][0]
    return dict(slab=slab, olen=olen, tail=tail, vecs=vecs, ntap=ntap, L=L)


_SPECS = (
    _fold_spec(22, 22, 3, 3, 1, 1, 0, 0, 1, 1, 9, 400),
    _fold_spec(17, 18, 2, 4, 2, 1, 2, 2, 1, 1, 8, 190),
    _fold_spec(5, 11, 2, 3, 1, 1, 2, 4, 1, 2, 6, 120),
)
_OGUARD = (12, 16, 8)


def _sc_fold_kernel(xh, yh, zh, oxh, oyh, ozh,
                    bx0, by0, bx1, by1, bz, slx0, sly0, slx1, sly1,
                    obx, oby, obz,
                    si0, si1, siz, so):
    wid = lax.axis_index("s") * _NC + lax.axis_index("c")
    g0 = wid * _GPW
    in_slots = ((bx0, by0), (bx1, by1))
    in_sems = (si0, si1)
    obufs = (obx, oby, obz)

    iota = lax.iota(jnp.int32, _LANES)
    mask_keys = sorted({(a, b)
                        for spec in _SPECS
                        for _, contribs in spec["vecs"]
                        for (_, a, b) in contribs if (a, b) != (0, _LANES)})
    masks = {ab: (iota >= ab[0]) & (iota < ab[1]) for ab in mask_keys}

    def in_copy(g, slot):
        for hbm, buf, spec in zip((xh, yh), in_slots[slot], _SPECS[:2]):
            nr = _B * spec["ntap"]
            yield pltpu.make_async_copy(
                hbm.at[pl.ds(g * nr, nr), :], buf, in_sems[slot])

    def z_in_copy(g):
        sz = _B * _SPECS[2]["slab"]
        return pltpu.make_async_copy(
            zh.at[pl.ds(g * sz, sz)], bz.at[pl.ds(0, sz)], siz)

    def out_copy(g):
        for hbm, buf, spec in zip((oxh, oyh, ozh), obufs, _SPECS):
            sz = _B * spec["olen"]
            yield pltpu.make_async_copy(
                buf.at[pl.ds(0, sz)], hbm.at[pl.ds(g * sz, sz)], so)

    slabs = ((slx0, sly0), (slx1, sly1))

    def stage_sample(slot, i, par):
        for buf2d, slab1d, spec in zip(in_slots[slot], slabs[par], _SPECS[:2]):
            ntap, L = spec["ntap"], spec["L"]
            for tap in range(ntap):
                row = i * ntap + tap
                dst = _PRE + tap * L
                for c in range(0, L - _LANES + 1, _LANES):
                    slab1d[pl.ds(dst + c, _LANES)] = buf2d[row, pl.ds(c, _LANES)]
                if L % _LANES:
                    slab1d[pl.ds(dst + L - _LANES, _LANES)] = (
                        buf2d[row, pl.ds(L - _LANES, _LANES)])

    def compute_sample(i, par):
        srcs = slabs[par] + (bz,)
        bases = (_PRE, _PRE, i * _SPECS[2]["slab"])
        for buf_i, base, buf_o, spec in zip(srcs, bases, obufs, _SPECS):
            obase = i * spec["olen"]
            pend = []

            def flush(pend):
                for o2, a2 in pend:
                    buf_o[pl.ds(obase + o2, _LANES)] = a2

            for off, contribs in spec["vecs"]:
                acc = None
                for lo, a, b in contribs:
                    v = buf_i[pl.ds(base + lo, _LANES)]
                    if (a, b) != (0, _LANES):
                        v = jnp.where(masks[(a, b)], v, 0.0)
                    acc = v if acc is None else acc + v
                pend.append((off, acc))
                if len(pend) == 4:
                    flush(pend)
                    pend = []
            flush(pend)

    for c in in_copy(g0, 0):
        c.start()
    for c in in_copy(g0 + 1, 1):
        c.start()
    z_in_copy(g0).start()

    @pl.loop(0, _GPW, step=2)
    def _(t):
        for slot in (0, 1):
            g = g0 + t + slot
            for c in in_copy(g, slot):
                c.wait()
            z_in_copy(g).wait()

            @pl.when(g > g0)
            def _():
                for c in out_copy(g - 1):
                    c.wait()

            stage_sample(slot, 0, 0)

            @pl.loop(0, _B, step=2)
            def _(i):
                stage_sample(slot, i + 1, 1)
                compute_sample(i, 0)

                @pl.when(i + 2 < _B)
                def _():
                    stage_sample(slot, i + 2, 0)

                compute_sample(i + 1, 1)

            for c in out_copy(g):
                c.start()

            @pl.when(g + 1 < g0 + _GPW)
            def _():
                z_in_copy(g + 1).start()

            @pl.when(t + slot + 2 < _GPW)
            def _():
                for c in in_copy(g + 2, slot):
                    c.start()

    for c in out_copy(g0 + _GPW - 1):
        c.wait()


@jax.jit
def kernel(x, y, z):
    run = pl.kernel(
        _sc_fold_kernel,
        out_type=tuple(
            jax.ShapeDtypeStruct((_ROWS * s["olen"],), jnp.float32)
            for s in _SPECS),
        mesh=plsc.VectorSubcoreMesh(core_axis_name="c", subcore_axis_name="s"),
        scratch_types=(
            pltpu.VMEM((_B * 9, 400), jnp.float32),
            pltpu.VMEM((_B * 8, 190), jnp.float32),
            pltpu.VMEM((_B * 9, 400), jnp.float32),
            pltpu.VMEM((_B * 8, 190), jnp.float32),
            pltpu.VMEM((_B * _SPECS[2]["slab"] + _LANES,), jnp.float32),
            pltpu.VMEM((_PRE + _SPECS[0]["slab"] + _LANES,), jnp.float32),
            pltpu.VMEM((_PRE + _SPECS[1]["slab"] + _LANES,), jnp.float32),
            pltpu.VMEM((_PRE + _SPECS[0]["slab"] + _LANES,), jnp.float32),
            pltpu.VMEM((_PRE + _SPECS[1]["slab"] + _LANES,), jnp.float32),
            pltpu.VMEM((_B * _SPECS[0]["olen"] + _OGUARD[0],), jnp.float32),
            pltpu.VMEM((_B * _SPECS[1]["olen"] + _OGUARD[1],), jnp.float32),
            pltpu.VMEM((_B * _SPECS[2]["olen"] + _OGUARD[2],), jnp.float32),
            pltpu.SemaphoreType.DMA,
            pltpu.SemaphoreType.DMA,
            pltpu.SemaphoreType.DMA,
            pltpu.SemaphoreType.DMA,
        ),
    )
    xo, yo, zo = run(x.reshape(64 * 1152, 400),
                     y.reshape(64 * 1024, 190),
                     z.reshape(-1))
    return (xo.reshape(64, 128, 22, 22),
            yo.reshape(64, 128, 17, 18),
            zo.reshape(64, 128, 5, 11))


# speedup vs baseline: 1.8026x; 1.8026x over previous
import jax
import jax.numpy as jnp
from jax import lax
from jax.experimental import pallas as pl
from jax.experimental.pallas import tpu as pltpu
from jax.experimental.pallas import tpu_sc as plsc

_LANES = 16
_NC, _NS = 2, 16
_NW = _NC * _NS
_ROWS = 64 * 128
_B = 8
_GROUPS = _ROWS // _B
_GPW = _GROUPS // _NW
_HDR = 16


def _fold_spec(oh, ow, kh, kw, sh, sw, ph, pw, dh, dw, flat_out):
    assert sw == 1, "all three folds have unit output-width stride"
    Lh = (oh + 2 * ph - dh * (kh - 1) - 1) // sh + 1
    Lw = (ow + 2 * pw - dw * (kw - 1) - 1) // sw + 1
    slab = kh * kw * Lh * Lw
    rows = [[] for _ in range(oh)]
    for ki in range(kh):
        for kj in range(kw):
            for lh in range(Lh):
                r = lh * sh + ki * dh - ph
                if r < 0 or r >= oh:
                    continue
                c0 = kj * dw - pw
                s = max(0, c0)
                e = min(Lw + c0, ow)
                if e > s:
                    rows[r].append((((ki * kw + kj) * Lh + lh) * Lw - c0, s, e))
    ks = (0,) if ow <= _LANES else (0, ow - _LANES)
    vecs = []
    for r in range(oh):
        for k in ks:
            contribs = []
            for src0, s, e in rows[r]:
                a = max(s - k, 0)
                b = min(e - k, _LANES)
                if b > a:
                    contribs.append((src0 + k, a, b))
            assert contribs
            vecs.append((r, k, contribs))
    return dict(slab=slab, oh=oh, ow=ow, olen=oh * ow, vecs=vecs,
                flat_out=flat_out)


_SPECS = (
    _fold_spec(22, 22, 3, 3, 1, 1, 0, 0, 1, 1, False),
    _fold_spec(17, 18, 2, 4, 2, 1, 2, 2, 1, 1, False),
    _fold_spec(5, 11, 2, 3, 1, 1, 2, 4, 1, 2, True),
)


def _sc_fold_kernel(xh, yh, zh, oxh, oyh, ozh,
                    ix0, iy0, iz0, ix1, iy1, iz1,
                    obx, oby, obz,
                    si0, si1, so):
    wid = lax.axis_index("s") * _NC + lax.axis_index("c")
    g0 = wid * _GPW
    in_slots = ((ix0, iy0, iz0), (ix1, iy1, iz1))
    in_sems = (si0, si1)
    obufs = (obx, oby, obz)

    iota = lax.iota(jnp.int32, _LANES)
    mask_keys = sorted({(a, b)
                        for spec in _SPECS
                        for _, _, contribs in spec["vecs"]
                        for (_, a, b) in contribs if (a, b) != (0, _LANES)})
    masks = {ab: (iota >= ab[0]) & (iota < ab[1]) for ab in mask_keys}

    def in_copy(g, slot):
        for hbm, buf, spec in zip((xh, yh, zh), in_slots[slot], _SPECS):
            sz = _B * spec["slab"]
            yield pltpu.make_async_copy(
                hbm.at[pl.ds(g * sz, sz)],
                buf.at[pl.ds(_HDR, sz)], in_sems[slot])

    def wave_copies(row0):
        for hbm, buf in ((oxh, obx), (oyh, oby)):
            yield pltpu.make_async_copy(
                buf, hbm.at[pl.ds(row0, 2), :, :], so)

    def z_out_copy(g):
        sz = _B * _SPECS[2]["olen"]
        return pltpu.make_async_copy(
            obz.at[pl.ds(0, sz)], ozh.at[pl.ds(g * sz, sz)], so)

    def compute_sample(bufs_i, i):
        for buf_i, buf_o, spec in zip(bufs_i, obufs, _SPECS):
            base = _HDR + i * spec["slab"]
            pend = []

            def flush(pend):
                for r, k, acc in pend:
                    if spec["flat_out"]:
                        buf_o[pl.ds(i * spec["olen"] + r * spec["ow"],
                                    _LANES)] = acc
                    else:
                        buf_o[i % 2, r, pl.ds(k, _LANES)] = acc

            for r, k, contribs in spec["vecs"]:
                acc = None
                for lo, a, b in contribs:
                    v = buf_i[pl.ds(base + lo, _LANES)]
                    if (a, b) != (0, _LANES):
                        v = jnp.where(masks[(a, b)], v, 0.0)
                    acc = v if acc is None else acc + v
                pend.append((r, k, acc))
                if len(pend) == 4:
                    flush(pend)
                    pend = []
            flush(pend)

    for c in in_copy(g0, 0):
        c.start()
    for c in in_copy(g0 + 1, 1):
        c.start()

    @pl.loop(0, _GPW, step=2)
    def _(t):
        for slot in (0, 1):
            g = g0 + t + slot
            for c in in_copy(g, slot):
                c.wait()
            bufs_i = in_slots[slot]

            @pl.loop(0, _B)
            def _(i):
                @pl.when((i % 2 == 0) & ((i > 0) | (g > g0)))
                def _():
                    for c in wave_copies(0):
                        c.wait()

                @pl.when((i == 0) & (g > g0))
                def _():
                    z_out_copy(0).wait()

                compute_sample(bufs_i, i)

                @pl.when(i % 2 == 1)
                def _():
                    for c in wave_copies(g * _B + i - 1):
                        c.start()

                @pl.when(i == _B - 1)
                def _():
                    z_out_copy(g).start()

            @pl.when(t + slot + 2 < _GPW)
            def _():
                for c in in_copy(g + 2, slot):
                    c.start()

    for c in wave_copies(0):
        c.wait()
    z_out_copy(0).wait()


@jax.jit
def kernel(x, y, z):
    sx, sy, sz = (s["slab"] for s in _SPECS)
    run = pl.kernel(
        _sc_fold_kernel,
        out_type=(
            jax.ShapeDtypeStruct((_ROWS, 22, 22), jnp.float32),
            jax.ShapeDtypeStruct((_ROWS, 17, 18), jnp.float32),
            jax.ShapeDtypeStruct((_ROWS * 55,), jnp.float32),
        ),
        mesh=plsc.VectorSubcoreMesh(core_axis_name="c", subcore_axis_name="s"),
        scratch_types=(
            pltpu.VMEM((_B * sx + 2 * _HDR,), jnp.float32),
            pltpu.VMEM((_B * sy + 2 * _HDR,), jnp.float32),
            pltpu.VMEM((_B * sz + 2 * _HDR,), jnp.float32),
            pltpu.VMEM((_B * sx + 2 * _HDR,), jnp.float32),
            pltpu.VMEM((_B * sy + 2 * _HDR,), jnp.float32),
            pltpu.VMEM((_B * sz + 2 * _HDR,), jnp.float32),
            pltpu.VMEM((2, 22, 22), jnp.float32),
            pltpu.VMEM((2, 17, 18), jnp.float32),
            pltpu.VMEM((_B * 55 + _HDR,), jnp.float32),
            pltpu.SemaphoreType.DMA,
            pltpu.SemaphoreType.DMA,
            pltpu.SemaphoreType.DMA,
        ),
    )
    xo, yo, zo = run(x.reshape(-1), y.reshape(-1), z.reshape(-1))
    return (xo.reshape(64, 128, 22, 22),
            yo.reshape(64, 128, 17, 18),
            zo.reshape(64, 128, 5, 11))


# speedup vs baseline: 1.8962x; 1.0519x over previous
import jax
import jax.numpy as jnp
from jax import lax
from jax.experimental import pallas as pl
from jax.experimental.pallas import tpu as pltpu
from jax.experimental.pallas import tpu_sc as plsc

_LANES = 16
_NC, _NS = 2, 16
_NW = _NC * _NS
_ROWS = 64 * 128
_B = 8
_GROUPS = _ROWS // _B
_GPW = _GROUPS // _NW
_SLOT = 48


def _fold_spec(oh, ow, kh, kw, sh, sw, ph, pw, dh, dw, flat_out):
    assert sw == 1
    Lh = (oh + 2 * ph - dh * (kh - 1) - 1) // sh + 1
    Lw = (ow + 2 * pw - dw * (kw - 1) - 1) // sw + 1
    L = Lh * Lw
    ntap = kh * kw
    rows = [[] for _ in range(oh)]
    for ki in range(kh):
        for kj in range(kw):
            for lh in range(Lh):
                r = lh * sh + ki * dh - ph
                if r < 0 or r >= oh:
                    continue
                c0 = kj * dw - pw
                s = max(0, c0)
                e = min(Lw + c0, ow)
                if e > s:
                    rows[r].append((ki * kw + kj, lh, s, e, c0))
    ks = (0,) if ow <= _LANES else (0, ow - _LANES)
    side = {}
    vecs = []
    for r in range(oh):
        for k in ks:
            contribs = []
            for tap, lh, s, e, c0 in rows[r]:
                a = max(s - k, 0)
                b = min(e - k, _LANES)
                if b <= a:
                    continue
                col = lh * Lw + k - c0
                if col < 0 or col + _LANES > L:
                    base = 0 if col < 0 else L - _LANES
                    slot = side.setdefault((tap, base), len(side))
                    contribs.append((None, slot * _SLOT + 16 + col - base, a, b))
                else:
                    contribs.append((tap, col, a, b))
            assert contribs
            vecs.append((r, k, contribs))
    return dict(L=L, ntap=ntap, oh=oh, ow=ow, olen=oh * ow, vecs=vecs,
                side=sorted(side.items(), key=lambda kv: kv[1]),
                flat_out=flat_out)


_SPECS = (
    _fold_spec(22, 22, 3, 3, 1, 1, 0, 0, 1, 1, False),
    _fold_spec(17, 18, 2, 4, 2, 1, 2, 2, 1, 1, False),
    _fold_spec(5, 11, 2, 3, 1, 1, 2, 4, 1, 2, True),
)


def _sc_fold_kernel(xh, yh, zh, oxh, oyh, ozh,
                    ix0, iy0, ix1, iy1, izb,
                    obx, oby, obz, sbx, sby, sbz,
                    si0, si1, siz, so):
    wid = lax.axis_index("s") * _NC + lax.axis_index("c")
    g0 = wid * _GPW
    in_slots = ((ix0, iy0), (ix1, iy1))
    in_sems = (si0, si1)
    obufs = (obx, oby, obz)
    sbufs = (sbx, sby, sbz)

    iota = lax.iota(jnp.int32, _LANES)
    mask_keys = sorted({(a, b)
                        for spec in _SPECS
                        for _, _, contribs in spec["vecs"]
                        for (_, _, a, b) in contribs if (a, b) != (0, _LANES)})
    masks = {ab: (iota >= ab[0]) & (iota < ab[1]) for ab in mask_keys}

    def in_copy(g, slot):
        for hbm, buf, spec in zip((xh, yh), in_slots[slot], _SPECS[:2]):
            nr = _B * spec["ntap"]
            yield pltpu.make_async_copy(
                hbm.at[pl.ds(g * nr, nr), :], buf, in_sems[slot])

    def z_in_copy(g):
        nr = _B * _SPECS[2]["ntap"]
        return pltpu.make_async_copy(zh.at[pl.ds(g * nr, nr), :], izb, siz)

    def wave_copies(row0):
        for hbm, buf in ((oxh, obx), (oyh, oby)):
            yield pltpu.make_async_copy(
                buf, hbm.at[pl.ds(row0, 2), :, :], so)

    def z_out_copy(g):
        sz = _B * _SPECS[2]["olen"]
        return pltpu.make_async_copy(
            obz.at[pl.ds(0, sz)], ozh.at[pl.ds(g * sz, sz)], so)

    def compute_sample(bufs_i, i):
        for buf_i, buf_o, sbuf, spec in zip(bufs_i, obufs, sbufs, _SPECS):
            ntap = spec["ntap"]
            for (tap, base), slot_i in spec["side"]:
                sbuf[pl.ds(slot_i * _SLOT + 16, _LANES)] = (
                    buf_i[i * ntap + tap, pl.ds(base, _LANES)])
            pend = []

            def flush(pend):
                for r, k, acc in pend:
                    if spec["flat_out"]:
                        buf_o[pl.ds(i * spec["olen"] + r * spec["ow"],
                                    _LANES)] = acc
                    else:
                        buf_o[i % 2, r, pl.ds(k, _LANES)] = acc

            for r, k, contribs in spec["vecs"]:
                acc = None
                for tap, col, a, b in contribs:
                    if tap is None:
                        v = sbuf[pl.ds(col, _LANES)]
                    else:
                        v = buf_i[i * ntap + tap, pl.ds(col, _LANES)]
                    if (a, b) != (0, _LANES):
                        v = jnp.where(masks[(a, b)], v, 0.0)
                    acc = v if acc is None else acc + v
                pend.append((r, k, acc))
                if len(pend) == 4:
                    flush(pend)
                    pend = []
            flush(pend)

    for c in in_copy(g0, 0):
        c.start()
    for c in in_copy(g0 + 1, 1):
        c.start()
    z_in_copy(g0).start()

    @pl.loop(0, _GPW, step=2)
    def _(t):
        for slot in (0, 1):
            g = g0 + t + slot
            for c in in_copy(g, slot):
                c.wait()
            z_in_copy(g).wait()
            bufs_i = (in_slots[slot][0], in_slots[slot][1], izb)

            @pl.loop(0, _B)
            def _(i):
                @pl.when((i % 2 == 0) & ((i > 0) | (g > g0)))
                def _():
                    for c in wave_copies(0):
                        c.wait()

                @pl.when((i == 0) & (g > g0))
                def _():
                    z_out_copy(0).wait()

                compute_sample(bufs_i, i)

                @pl.when(i % 2 == 1)
                def _():
                    for c in wave_copies(g * _B + i - 1):
                        c.start()

                @pl.when(i == _B - 1)
                def _():
                    z_out_copy(g).start()

            @pl.when(g + 1 < g0 + _GPW)
            def _():
                z_in_copy(g + 1).start()

            @pl.when(t + slot + 2 < _GPW)
            def _():
                for c in in_copy(g + 2, slot):
                    c.start()

    for c in wave_copies(0):
        c.wait()
    z_out_copy(0).wait()


@jax.jit
def kernel(x, y, z):
    x2 = x.reshape(64 * 1152, 400)
    y2 = y.reshape(64 * 1024, 190)
    z2 = z.reshape(64 * 768, 120)
    run = pl.kernel(
        _sc_fold_kernel,
        out_type=(
            jax.ShapeDtypeStruct((_ROWS, 22, 22), jnp.float32),
            jax.ShapeDtypeStruct((_ROWS, 17, 18), jnp.float32),
            jax.ShapeDtypeStruct((_ROWS * 55,), jnp.float32),
        ),
        mesh=plsc.VectorSubcoreMesh(core_axis_name="c", subcore_axis_name="s"),
        scratch_types=(
            pltpu.VMEM((_B * 9, 400), jnp.float32),
            pltpu.VMEM((_B * 8, 190), jnp.float32),
            pltpu.VMEM((_B * 9, 400), jnp.float32),
            pltpu.VMEM((_B * 8, 190), jnp.float32),
            pltpu.VMEM((_B * 6, 120), jnp.float32),
            pltpu.VMEM((2, 22, 22), jnp.float32),
            pltpu.VMEM((2, 17, 18), jnp.float32),
            pltpu.VMEM((_B * 55 + 16,), jnp.float32),
            pltpu.VMEM((max(len(_SPECS[0]["side"]), 1) * _SLOT,), jnp.float32),
            pltpu.VMEM((max(len(_SPECS[1]["side"]), 1) * _SLOT,), jnp.float32),
            pltpu.VMEM((max(len(_SPECS[2]["side"]), 1) * _SLOT,), jnp.float32),
            pltpu.SemaphoreType.DMA,
            pltpu.SemaphoreType.DMA,
            pltpu.SemaphoreType.DMA,
            pltpu.SemaphoreType.DMA,
        ),
    )
    xo, yo, zo = run(x2, y2, z2)
    return (xo.reshape(64, 128, 22, 22),
            yo.reshape(64, 128, 17, 18),
            zo.reshape(64, 128, 5, 11))


# speedup vs baseline: 2.1354x; 1.1262x over previous
import jax
import jax.numpy as jnp
from jax import lax
from jax.experimental import pallas as pl
from jax.experimental.pallas import tpu as pltpu
from jax.experimental.pallas import tpu_sc as plsc

_LANES = 16
_NC, _NS = 2, 16
_NW = _NC * _NS
_ROWS = 64 * 128
_B = 8
_GROUPS = _ROWS // _B
_GPW = _GROUPS // _NW
_SLOT = 48


def _fold_spec(oh, ow, kh, kw, sh, sw, ph, pw, dh, dw, flat_out):
    assert sw == 1
    Lh = (oh + 2 * ph - dh * (kh - 1) - 1) // sh + 1
    Lw = (ow + 2 * pw - dw * (kw - 1) - 1) // sw + 1
    L = Lh * Lw
    ntap = kh * kw
    rows = [[] for _ in range(oh)]
    for ki in range(kh):
        for kj in range(kw):
            for lh in range(Lh):
                r = lh * sh + ki * dh - ph
                if r < 0 or r >= oh:
                    continue
                c0 = kj * dw - pw
                s = max(0, c0)
                e = min(Lw + c0, ow)
                if e > s:
                    rows[r].append((ki * kw + kj, lh, s, e, c0))
    ks = (0,) if ow <= _LANES else (0, ow - _LANES)
    side = {}
    vecs = []
    for r in range(oh):
        for k in ks:
            contribs = []
            for tap, lh, s, e, c0 in rows[r]:
                a = max(s - k, 0)
                b = min(e - k, _LANES)
                if b <= a:
                    continue
                col = lh * Lw + k - c0
                if col < 0 or col + _LANES > L:
                    base = 0 if col < 0 else L - _LANES
                    slot = side.setdefault((tap, base), len(side))
                    contribs.append((None, slot * _SLOT + 16 + col - base, a, b))
                else:
                    contribs.append((tap, col, a, b))
            assert contribs
            vecs.append((r, k, contribs))
    return dict(L=L, ntap=ntap, oh=oh, ow=ow, olen=oh * ow, vecs=vecs,
                side=sorted(side.items(), key=lambda kv: kv[1]),
                flat_out=flat_out)


_SPECS = (
    _fold_spec(22, 22, 3, 3, 1, 1, 0, 0, 1, 1, False),
    _fold_spec(17, 18, 2, 4, 2, 1, 2, 2, 1, 1, False),
    _fold_spec(5, 11, 2, 3, 1, 1, 2, 4, 1, 2, True),
)


def _sc_fold_kernel(xh, yh, zh, oxh, oyh, ozh,
                    ix0, iy0, ix1, iy1, izb,
                    obx, oby, obz, sbx, sby, sbz,
                    si0, si1, siz, so):
    wid = lax.axis_index("s") * _NC + lax.axis_index("c")
    g0 = wid * _GPW
    in_slots = ((ix0, iy0), (ix1, iy1))
    in_sems = (si0, si1)
    obufs = (obx, oby, obz)
    sbufs = (sbx, sby, sbz)

    iota = lax.iota(jnp.int32, _LANES)
    mask_keys = sorted({(a, b)
                        for spec in _SPECS
                        for _, _, contribs in spec["vecs"]
                        for (_, _, a, b) in contribs if (a, b) != (0, _LANES)})
    masks = {ab: (iota >= ab[0]) & (iota < ab[1]) for ab in mask_keys}

    def in_copy(g, slot):
        for hbm, buf, spec in zip((xh, yh), in_slots[slot], _SPECS[:2]):
            nr = _B * spec["ntap"]
            yield pltpu.make_async_copy(
                hbm.at[pl.ds(g * nr, nr), :], buf, in_sems[slot])

    def z_in_copy(g):
        nr = _B * _SPECS[2]["ntap"]
        return pltpu.make_async_copy(zh.at[pl.ds(g * nr, nr), :], izb, siz)

    def wave_copies(row0):
        for hbm, buf in ((oxh, obx), (oyh, oby)):
            yield pltpu.make_async_copy(
                buf, hbm.at[pl.ds(row0, 2), :, :], so)

    def z_out_copy(g):
        sz = _B * _SPECS[2]["olen"]
        return pltpu.make_async_copy(
            obz.at[pl.ds(0, sz)], ozh.at[pl.ds(g * sz, sz)], so)

    def compute_sample(bufs_i, i):
        for buf_i, buf_o, sbuf, spec in zip(bufs_i, obufs, sbufs, _SPECS):
            ntap = spec["ntap"]
            rb = i * ntap
            for (tap, base), slot_i in spec["side"]:
                sbuf[pl.ds(slot_i * _SLOT + 16, _LANES)] = (
                    buf_i[rb + tap, pl.ds(base, _LANES)])
            pend = []

            def flush(pend):
                for r, k, acc in pend:
                    if spec["flat_out"]:
                        buf_o[pl.ds(i * spec["olen"] + r * spec["ow"],
                                    _LANES)] = acc
                    else:
                        buf_o[i % 2, r, pl.ds(k, _LANES)] = acc

            for r, k, contribs in spec["vecs"]:
                acc = None
                for tap, col, a, b in contribs:
                    if tap is None:
                        v = sbuf[pl.ds(col, _LANES)]
                    else:
                        v = buf_i[rb + tap, pl.ds(col, _LANES)]
                    if (a, b) != (0, _LANES):
                        v = jnp.where(masks[(a, b)], v, 0.0)
                    acc = v if acc is None else acc + v
                pend.append((r, k, acc))
                if len(pend) == 8:
                    flush(pend)
                    pend = []
            flush(pend)

    for c in in_copy(g0, 0):
        c.start()
    for c in in_copy(g0 + 1, 1):
        c.start()
    z_in_copy(g0).start()

    @pl.loop(0, _GPW, step=2)
    def _(t):
        for slot in (0, 1):
            g = g0 + t + slot
            for c in in_copy(g, slot):
                c.wait()
            z_in_copy(g).wait()
            bufs_i = (in_slots[slot][0], in_slots[slot][1], izb)

            @pl.loop(0, _B)
            def _(i):
                @pl.when((i % 2 == 0) & ((i > 0) | (g > g0)))
                def _():
                    for c in wave_copies(0):
                        c.wait()

                @pl.when((i == 0) & (g > g0))
                def _():
                    z_out_copy(0).wait()

                compute_sample(bufs_i, i)

                @pl.when(i % 2 == 1)
                def _():
                    for c in wave_copies(g * _B + i - 1):
                        c.start()

                @pl.when(i == _B - 1)
                def _():
                    z_out_copy(g).start()

            @pl.when(g + 1 < g0 + _GPW)
            def _():
                z_in_copy(g + 1).start()

            @pl.when(t + slot + 2 < _GPW)
            def _():
                for c in in_copy(g + 2, slot):
                    c.start()

    for c in wave_copies(0):
        c.wait()
    z_out_copy(0).wait()


@jax.jit
def kernel(x, y, z):
    x2 = x.reshape(64 * 1152, 400)
    y2 = y.reshape(64 * 1024, 190)
    z2 = z.reshape(64 * 768, 120)
    run = pl.kernel(
        _sc_fold_kernel,
        out_type=(
            jax.ShapeDtypeStruct((_ROWS, 22, 22), jnp.float32),
            jax.ShapeDtypeStruct((_ROWS, 17, 18), jnp.float32),
            jax.ShapeDtypeStruct((_ROWS * 55,), jnp.float32),
        ),
        mesh=plsc.VectorSubcoreMesh(core_axis_name="c", subcore_axis_name="s"),
        scratch_types=(
            pltpu.VMEM((_B * 9, 400), jnp.float32),
            pltpu.VMEM((_B * 8, 190), jnp.float32),
            pltpu.VMEM((_B * 9, 400), jnp.float32),
            pltpu.VMEM((_B * 8, 190), jnp.float32),
            pltpu.VMEM((_B * 6, 120), jnp.float32),
            pltpu.VMEM((2, 22, 22), jnp.float32),
            pltpu.VMEM((2, 17, 18), jnp.float32),
            pltpu.VMEM((_B * 55 + 16,), jnp.float32),
            pltpu.VMEM((max(len(_SPECS[0]["side"]), 1) * _SLOT,), jnp.float32),
            pltpu.VMEM((max(len(_SPECS[1]["side"]), 1) * _SLOT,), jnp.float32),
            pltpu.VMEM((max(len(_SPECS[2]["side"]), 1) * _SLOT,), jnp.float32),
            pltpu.SemaphoreType.DMA,
            pltpu.SemaphoreType.DMA,
            pltpu.SemaphoreType.DMA,
            pltpu.SemaphoreType.DMA,
        ),
    )
    xo, yo, zo = run(x2, y2, z2)
    return (xo.reshape(64, 128, 22, 22),
            yo.reshape(64, 128, 17, 18),
            zo.reshape(64, 128, 5, 11))


# speedup vs baseline: 2.1698x; 1.0161x over previous
import jax
import jax.numpy as jnp
from jax import lax
from jax.experimental import pallas as pl
from jax.experimental.pallas import tpu as pltpu
from jax.experimental.pallas import tpu_sc as plsc

_LANES = 16
_NC, _NS = 2, 16
_NW = _NC * _NS
_ROWS = 64 * 128
_B = 8
_GROUPS = _ROWS // _B
_GPW = _GROUPS // _NW
_SLOT = 48


def _fold_spec(oh, ow, kh, kw, sh, sw, ph, pw, dh, dw, flat_out):
    assert sw == 1
    Lh = (oh + 2 * ph - dh * (kh - 1) - 1) // sh + 1
    Lw = (ow + 2 * pw - dw * (kw - 1) - 1) // sw + 1
    L = Lh * Lw
    ntap = kh * kw
    rows = [[] for _ in range(oh)]
    for ki in range(kh):
        for kj in range(kw):
            for lh in range(Lh):
                r = lh * sh + ki * dh - ph
                if r < 0 or r >= oh:
                    continue
                c0 = kj * dw - pw
                s = max(0, c0)
                e = min(Lw + c0, ow)
                if e > s:
                    rows[r].append((ki * kw + kj, lh, s, e, c0))
    ks = (0,) if ow <= _LANES else (0, ow - _LANES)
    side = {}
    vecs = []
    for r in range(oh):
        for k in ks:
            contribs = []
            for tap, lh, s, e, c0 in rows[r]:
                a = max(s - k, 0)
                b = min(e - k, _LANES)
                if b <= a:
                    continue
                col = lh * Lw + k - c0
                if col < 0 or col + _LANES > L:
                    base = 0 if col < 0 else L - _LANES
                    slot = side.setdefault((tap, base), len(side))
                    contribs.append((None, slot * _SLOT + 16 + col - base, a, b))
                else:
                    contribs.append((tap, col, a, b))
            assert contribs
            vecs.append((r, k, contribs))
    return dict(L=L, ntap=ntap, oh=oh, ow=ow, olen=oh * ow, vecs=vecs,
                side=sorted(side.items(), key=lambda kv: kv[1]),
                flat_out=flat_out)


_SPECS = (
    _fold_spec(22, 22, 3, 3, 1, 1, 0, 0, 1, 1, False),
    _fold_spec(17, 18, 2, 4, 2, 1, 2, 2, 1, 1, False),
    _fold_spec(5, 11, 2, 3, 1, 1, 2, 4, 1, 2, True),
)


def _sc_fold_kernel(xh, yh, zh, oxh, oyh, ozh,
                    ix0, iy0, ix1, iy1, izb,
                    obx, oby, obz, sbx, sby, sbz,
                    si0, si1, siz, so):
    wid = lax.axis_index("s") * _NC + lax.axis_index("c")
    g0 = wid * _GPW
    in_slots = ((ix0, iy0), (ix1, iy1))
    in_sems = (si0, si1)
    obufs = (obx, oby, obz)
    sbufs = (sbx, sby, sbz)

    iota = lax.iota(jnp.int32, _LANES)
    mask_keys = sorted({(a, b)
                        for spec in _SPECS
                        for _, _, contribs in spec["vecs"]
                        for (_, _, a, b) in contribs if (a, b) != (0, _LANES)})
    masks = {ab: (iota >= ab[0]) & (iota < ab[1]) for ab in mask_keys}

    def in_copy(g, slot):
        for hbm, buf, spec in zip((xh, yh), in_slots[slot], _SPECS[:2]):
            nr = _B * spec["ntap"]
            yield pltpu.make_async_copy(
                hbm.at[pl.ds(g * nr, nr), :], buf, in_sems[slot])

    def z_in_copy(g):
        nr = _B * _SPECS[2]["ntap"]
        return pltpu.make_async_copy(zh.at[pl.ds(g * nr, nr), :], izb, siz)

    def wave_copies(row0):
        for hbm, buf in ((oxh, obx), (oyh, oby)):
            yield pltpu.make_async_copy(
                buf, hbm.at[pl.ds(row0, 2), :, :], so)

    def z_out_copy(g):
        sz = _B * _SPECS[2]["olen"]
        return pltpu.make_async_copy(
            obz.at[pl.ds(0, sz)], ozh.at[pl.ds(g * sz, sz)], so)

    def compute_sample(bufs_i, i):
        for buf_i, buf_o, sbuf, spec in zip(bufs_i, obufs, sbufs, _SPECS):
            ntap = spec["ntap"]
            rb = i * ntap
            for (tap, base), slot_i in spec["side"]:
                sbuf[pl.ds(slot_i * _SLOT + 16, _LANES)] = (
                    buf_i[rb + tap, pl.ds(base, _LANES)])
            pend = []

            def flush(pend):
                for r, k, acc in pend:
                    if spec["flat_out"]:
                        buf_o[pl.ds(i * spec["olen"] + r * spec["ow"],
                                    _LANES)] = acc
                    else:
                        buf_o[i % 2, r, pl.ds(k, _LANES)] = acc

            for r, k, contribs in spec["vecs"]:
                acc = None
                for tap, col, a, b in contribs:
                    if tap is None:
                        v = sbuf[pl.ds(col, _LANES)]
                    else:
                        v = buf_i[rb + tap, pl.ds(col, _LANES)]
                    if (a, b) != (0, _LANES):
                        v = jnp.where(masks[(a, b)], v, 0.0)
                    acc = v if acc is None else acc + v
                pend.append((r, k, acc))
                if len(pend) == 16:
                    flush(pend)
                    pend = []
            flush(pend)

    for c in in_copy(g0, 0):
        c.start()
    for c in in_copy(g0 + 1, 1):
        c.start()
    z_in_copy(g0).start()

    @pl.loop(0, _GPW, step=2)
    def _(t):
        for slot in (0, 1):
            g = g0 + t + slot
            for c in in_copy(g, slot):
                c.wait()
            z_in_copy(g).wait()
            bufs_i = (in_slots[slot][0], in_slots[slot][1], izb)

            @pl.loop(0, _B)
            def _(i):
                @pl.when((i % 2 == 0) & ((i > 0) | (g > g0)))
                def _():
                    for c in wave_copies(0):
                        c.wait()

                @pl.when((i == 0) & (g > g0))
                def _():
                    z_out_copy(0).wait()

                compute_sample(bufs_i, i)

                @pl.when(i % 2 == 1)
                def _():
                    for c in wave_copies(g * _B + i - 1):
                        c.start()

                @pl.when(i == _B - 1)
                def _():
                    z_out_copy(g).start()

            @pl.when(g + 1 < g0 + _GPW)
            def _():
                z_in_copy(g + 1).start()

            @pl.when(t + slot + 2 < _GPW)
            def _():
                for c in in_copy(g + 2, slot):
                    c.start()

    for c in wave_copies(0):
        c.wait()
    z_out_copy(0).wait()


@jax.jit
def kernel(x, y, z):
    x2 = x.reshape(64 * 1152, 400)
    y2 = y.reshape(64 * 1024, 190)
    z2 = z.reshape(64 * 768, 120)
    run = pl.kernel(
        _sc_fold_kernel,
        out_type=(
            jax.ShapeDtypeStruct((_ROWS, 22, 22), jnp.float32),
            jax.ShapeDtypeStruct((_ROWS, 17, 18), jnp.float32),
            jax.ShapeDtypeStruct((_ROWS * 55,), jnp.float32),
        ),
        mesh=plsc.VectorSubcoreMesh(core_axis_name="c", subcore_axis_name="s"),
        scratch_types=(
            pltpu.VMEM((_B * 9, 400), jnp.float32),
            pltpu.VMEM((_B * 8, 190), jnp.float32),
            pltpu.VMEM((_B * 9, 400), jnp.float32),
            pltpu.VMEM((_B * 8, 190), jnp.float32),
            pltpu.VMEM((_B * 6, 120), jnp.float32),
            pltpu.VMEM((2, 22, 22), jnp.float32),
            pltpu.VMEM((2, 17, 18), jnp.float32),
            pltpu.VMEM((_B * 55 + 16,), jnp.float32),
            pltpu.VMEM((max(len(_SPECS[0]["side"]), 1) * _SLOT,), jnp.float32),
            pltpu.VMEM((max(len(_SPECS[1]["side"]), 1) * _SLOT,), jnp.float32),
            pltpu.VMEM((max(len(_SPECS[2]["side"]), 1) * _SLOT,), jnp.float32),
            pltpu.SemaphoreType.DMA,
            pltpu.SemaphoreType.DMA,
            pltpu.SemaphoreType.DMA,
            pltpu.SemaphoreType.DMA,
        ),
    )
    xo, yo, zo = run(x2, y2, z2)
    return (xo.reshape(64, 128, 22, 22),
            yo.reshape(64, 128, 17, 18),
            zo.reshape(64, 128, 5, 11))


# speedup vs baseline: 2.1854x; 1.0072x over previous
import jax
import jax.numpy as jnp
from jax import lax
from jax.experimental import pallas as pl
from jax.experimental.pallas import tpu as pltpu
from jax.experimental.pallas import tpu_sc as plsc

_LANES = 16
_NC, _NS = 2, 16
_NW = _NC * _NS
_ROWS = 64 * 128
_B = 8
_GROUPS = _ROWS // _B
_GPW = _GROUPS // _NW
_SLOT = 48


def _fold_spec(oh, ow, kh, kw, sh, sw, ph, pw, dh, dw, flat_out):
    assert sw == 1
    Lh = (oh + 2 * ph - dh * (kh - 1) - 1) // sh + 1
    Lw = (ow + 2 * pw - dw * (kw - 1) - 1) // sw + 1
    L = Lh * Lw
    ntap = kh * kw
    rows = [[] for _ in range(oh)]
    for ki in range(kh):
        for kj in range(kw):
            for lh in range(Lh):
                r = lh * sh + ki * dh - ph
                if r < 0 or r >= oh:
                    continue
                c0 = kj * dw - pw
                s = max(0, c0)
                e = min(Lw + c0, ow)
                if e > s:
                    rows[r].append((ki * kw + kj, lh, s, e, c0))
    ks = (0,) if ow <= _LANES else (0, ow - _LANES)
    side = {}
    vecs = []
    for r in range(oh):
        for k in ks:
            contribs = []
            for tap, lh, s, e, c0 in rows[r]:
                a = max(s - k, 0)
                b = min(e - k, _LANES)
                if b <= a:
                    continue
                col = lh * Lw + k - c0
                if col < 0 or col + _LANES > L:
                    base = 0 if col < 0 else L - _LANES
                    slot = side.setdefault((tap, base), len(side))
                    contribs.append((None, slot * _SLOT + 16 + col - base, a, b))
                else:
                    contribs.append((tap, col, a, b))
            assert contribs
            vecs.append((r, k, contribs))
    return dict(L=L, ntap=ntap, oh=oh, ow=ow, olen=oh * ow, vecs=vecs,
                side=sorted(side.items(), key=lambda kv: kv[1]),
                flat_out=flat_out)


_SPECS = (
    _fold_spec(22, 22, 3, 3, 1, 1, 0, 0, 1, 1, False),
    _fold_spec(17, 18, 2, 4, 2, 1, 2, 2, 1, 1, False),
    _fold_spec(5, 11, 2, 3, 1, 1, 2, 4, 1, 2, True),
)


def _sc_fold_kernel(xh, yh, zh, oxh, oyh, ozh,
                    ix0, iy0, ix1, iy1, izb,
                    obx, oby, obz, sbx, sby, sbz,
                    si0, si1, siz, so):
    wid = lax.axis_index("s") * _NC + lax.axis_index("c")
    g0 = wid * _GPW
    in_slots = ((ix0, iy0), (ix1, iy1))
    in_sems = (si0, si1)
    obufs = (obx, oby, obz)
    sbufs = (sbx, sby, sbz)

    iota = lax.iota(jnp.int32, _LANES)
    mask_keys = sorted({(a, b)
                        for spec in _SPECS
                        for _, _, contribs in spec["vecs"]
                        for (_, _, a, b) in contribs if (a, b) != (0, _LANES)})
    masks = {ab: (iota >= ab[0]) & (iota < ab[1]) for ab in mask_keys}

    def in_copy(g, slot):
        for hbm, buf, spec in zip((xh, yh), in_slots[slot], _SPECS[:2]):
            nr = _B * spec["ntap"]
            yield pltpu.make_async_copy(
                hbm.at[pl.ds(g * nr, nr), :], buf, in_sems[slot])

    def z_in_copy(g):
        nr = _B * _SPECS[2]["ntap"]
        return pltpu.make_async_copy(zh.at[pl.ds(g * nr, nr), :], izb, siz)

    def wave_copies(row0):
        for hbm, buf in ((oxh, obx), (oyh, oby)):
            yield pltpu.make_async_copy(
                buf, hbm.at[pl.ds(row0, 2), :, :], so)

    def z_out_copy(g):
        sz = _B * _SPECS[2]["olen"]
        return pltpu.make_async_copy(
            obz.at[pl.ds(0, sz)], ozh.at[pl.ds(g * sz, sz)], so)

    def compute_sample(bufs_i, i):
        for buf_i, buf_o, sbuf, spec in zip(bufs_i, obufs, sbufs, _SPECS):
            ntap = spec["ntap"]
            rb = i * ntap
            for (tap, base), slot_i in spec["side"]:
                sbuf[pl.ds(slot_i * _SLOT + 16, _LANES)] = (
                    buf_i[rb + tap, pl.ds(base, _LANES)])
            pend = []

            def flush(pend):
                for r, k, acc in pend:
                    if spec["flat_out"]:
                        buf_o[pl.ds(i * spec["olen"] + r * spec["ow"],
                                    _LANES)] = acc
                    else:
                        buf_o[i % 2, r, pl.ds(k, _LANES)] = acc

            for r, k, contribs in spec["vecs"]:
                acc = None
                for tap, col, a, b in contribs:
                    if tap is None:
                        v = sbuf[pl.ds(col, _LANES)]
                    else:
                        v = buf_i[rb + tap, pl.ds(col, _LANES)]
                    if (a, b) != (0, _LANES):
                        v = jnp.where(masks[(a, b)], v, 0.0)
                    acc = v if acc is None else acc + v
                pend.append((r, k, acc))
                if len(pend) == 24:
                    flush(pend)
                    pend = []
            flush(pend)

    for c in in_copy(g0, 0):
        c.start()
    for c in in_copy(g0 + 1, 1):
        c.start()
    z_in_copy(g0).start()

    @pl.loop(0, _GPW, step=2)
    def _(t):
        for slot in (0, 1):
            g = g0 + t + slot
            for c in in_copy(g, slot):
                c.wait()
            z_in_copy(g).wait()
            bufs_i = (in_slots[slot][0], in_slots[slot][1], izb)

            @pl.loop(0, _B)
            def _(i):
                @pl.when((i % 2 == 0) & ((i > 0) | (g > g0)))
                def _():
                    for c in wave_copies(0):
                        c.wait()

                @pl.when((i == 0) & (g > g0))
                def _():
                    z_out_copy(0).wait()

                compute_sample(bufs_i, i)

                @pl.when(i % 2 == 1)
                def _():
                    for c in wave_copies(g * _B + i - 1):
                        c.start()

                @pl.when(i == _B - 1)
                def _():
                    z_out_copy(g).start()

            @pl.when(g + 1 < g0 + _GPW)
            def _():
                z_in_copy(g + 1).start()

            @pl.when(t + slot + 2 < _GPW)
            def _():
                for c in in_copy(g + 2, slot):
                    c.start()

    for c in wave_copies(0):
        c.wait()
    z_out_copy(0).wait()


@jax.jit
def kernel(x, y, z):
    x2 = x.reshape(64 * 1152, 400)
    y2 = y.reshape(64 * 1024, 190)
    z2 = z.reshape(64 * 768, 120)
    run = pl.kernel(
        _sc_fold_kernel,
        out_type=(
            jax.ShapeDtypeStruct((_ROWS, 22, 22), jnp.float32),
            jax.ShapeDtypeStruct((_ROWS, 17, 18), jnp.float32),
            jax.ShapeDtypeStruct((_ROWS * 55,), jnp.float32),
        ),
        mesh=plsc.VectorSubcoreMesh(core_axis_name="c", subcore_axis_name="s"),
        scratch_types=(
            pltpu.VMEM((_B * 9, 400), jnp.float32),
            pltpu.VMEM((_B * 8, 190), jnp.float32),
            pltpu.VMEM((_B * 9, 400), jnp.float32),
            pltpu.VMEM((_B * 8, 190), jnp.float32),
            pltpu.VMEM((_B * 6, 120), jnp.float32),
            pltpu.VMEM((2, 22, 22), jnp.float32),
            pltpu.VMEM((2, 17, 18), jnp.float32),
            pltpu.VMEM((_B * 55 + 16,), jnp.float32),
            pltpu.VMEM((max(len(_SPECS[0]["side"]), 1) * _SLOT,), jnp.float32),
            pltpu.VMEM((max(len(_SPECS[1]["side"]), 1) * _SLOT,), jnp.float32),
            pltpu.VMEM((max(len(_SPECS[2]["side"]), 1) * _SLOT,), jnp.float32),
            pltpu.SemaphoreType.DMA,
            pltpu.SemaphoreType.DMA,
            pltpu.SemaphoreType.DMA,
            pltpu.SemaphoreType.DMA,
        ),
    )
    xo, yo, zo = run(x2, y2, z2)
    return (xo.reshape(64, 128, 22, 22),
            yo.reshape(64, 128, 17, 18),
            zo.reshape(64, 128, 5, 11))
